# speedup vs baseline: 10.1998x; 10.1998x over previous
import functools

import jax
import jax.numpy as jnp
from jax.experimental import pallas as pl
from jax.experimental.pallas import tpu as pltpu

_DIM = 1024
_NE = 8
_NG = 4
_BLK = 2048
_NTOK = 32768


def _gate_body(x_ref, wt_ref, w_out_ref, i_out_ref):
    x = x_ref[...]
    wt = wt_ref[...]
    raw = jax.lax.dot_general(
        x, wt, (((1,), (0,)), ((), ())),
        preferred_element_type=jnp.float32)
    s = jax.nn.sigmoid(raw)
    st = s.T

    e = jax.lax.broadcasted_iota(jnp.int32, (_NE, _BLK), 0)
    even = (e % 2) == 0
    partner = jnp.where(even,
                        jnp.roll(st, -1, axis=0),
                        jnp.roll(st, 1, axis=0))
    gm = jnp.maximum(st, partner)
    gidx = e // 2
    cnt = jnp.zeros((_NE, _BLK), jnp.int32)
    for k in (2, 4, 6):
        other = jnp.roll(gm, -k, axis=0)
        og = (gidx + (k // 2)) % _NG
        beats = (other > gm) | ((other == gm) & (og < gidx))
        cnt = cnt + beats.astype(jnp.int32)
    selected = cnt < 2
    neg = jnp.float32(-jnp.inf)
    masked = jnp.where(selected, st, neg)

    m0 = jnp.max(masked, axis=0, keepdims=True)
    is0 = masked == m0
    idx0 = jnp.min(jnp.where(is0, e, _NE), axis=0, keepdims=True)
    masked2 = jnp.where(e == idx0, neg, masked)
    m1 = jnp.max(masked2, axis=0, keepdims=True)
    is1 = masked2 == m1
    idx1 = jnp.min(jnp.where(is1, e, _NE), axis=0, keepdims=True)

    tot = m0 + m1
    w_out_ref[...] = jnp.concatenate([m0 / tot, m1 / tot], axis=0)
    i_out_ref[...] = jnp.concatenate([idx0, idx1], axis=0)


@jax.jit
def kernel(x, weight):
    n_tok = x.shape[0]
    grid = (n_tok // _BLK,)
    w_t, i_t = pl.pallas_call(
        _gate_body,
        grid=grid,
        in_specs=[
            pl.BlockSpec((_BLK, _DIM), lambda i: (i, 0)),
            pl.BlockSpec((_DIM, _NE), lambda i: (0, 0)),
        ],
        out_specs=[
            pl.BlockSpec((2, _BLK), lambda i: (0, i)),
            pl.BlockSpec((2, _BLK), lambda i: (0, i)),
        ],
        out_shape=[
            jax.ShapeDtypeStruct((2, n_tok), jnp.float32),
            jax.ShapeDtypeStruct((2, n_tok), jnp.int32),
        ],
    )(x, weight.T)
    return w_t.T, i_t.T
